# Initial kernel scaffold; baseline (speedup 1.0000x reference)
import jax
import jax.numpy as jnp
from jax.experimental import pallas as pl


def kernel(x, edge_index, edge_attr, W1, root1, b1, W2, root2, b2):
    raise NotImplementedError("write your pallas kernel here")


# speedup vs baseline: 8.0718x; 8.0718x over previous
import functools

import jax
import jax.numpy as jnp
from jax import lax
from jax.experimental import pallas as pl
from jax.experimental.pallas import tpu as pltpu
from jax.experimental.pallas import tpu_sc as plsc

N = 10000
E = 320000
G = 128
N_GROUPS = E // G
NC = 2
NS = 16
NW = NC * NS
N_PAD = 10240
RPS = N_PAD // NS

_F32 = jnp.float32


_BLK = 1000


def _prep1_body(x_ref, w_ref, g1_ref, r1_ref):
    y = jnp.dot(x_ref[...], w_ref[...], preferred_element_type=_F32)
    g1_ref[...] = y[:, 0:32]
    r1_ref[...] = y[:, 32:48]


def _tc_prep1(x, wcat1):
    return pl.pallas_call(
        _prep1_body,
        grid=(N // _BLK,),
        in_specs=[
            pl.BlockSpec((_BLK, 128), lambda i: (i, 0)),
            pl.BlockSpec((128, 48), lambda i: (0, 0)),
        ],
        out_specs=[
            pl.BlockSpec((_BLK, 32), lambda i: (i, 0)),
            pl.BlockSpec((_BLK, 16), lambda i: (i, 0)),
        ],
        out_shape=[
            jax.ShapeDtypeStruct((N, 32), _F32),
            jax.ShapeDtypeStruct((N, 16), _F32),
        ],
    )(x, wcat1)


def _mid_body(accp_ref, r1_ref, b1_ref, wg2_ref, wr2_ref, g2_ref, r2_ref, di_ref):
    a = accp_ref[0] + accp_ref[1]
    agg = a[:, 0:16]
    deg = jnp.maximum(a[:, 16:17], 1.0)
    v = agg / deg + r1_ref[...] + b1_ref[...]
    h = jnp.where(v > 0, v, jnp.exp(v) - 1.0)
    g2_ref[...] = jnp.dot(h, wg2_ref[...], preferred_element_type=_F32)
    r2_ref[...] = jnp.dot(h, wr2_ref[...], preferred_element_type=_F32)
    di_ref[...] = jnp.broadcast_to(1.0 / deg, deg.shape[:1] + (8,))


def _tc_mid(acc_p, r1, b1row, wg2, wr2):
    return pl.pallas_call(
        _mid_body,
        grid=(N // _BLK,),
        in_specs=[
            pl.BlockSpec((2, _BLK, 32), lambda i: (0, i, 0)),
            pl.BlockSpec((_BLK, 16), lambda i: (i, 0)),
            pl.BlockSpec((1, 16), lambda i: (0, 0)),
            pl.BlockSpec((16, 16), lambda i: (0, 0)),
            pl.BlockSpec((16, 8), lambda i: (0, 0)),
        ],
        out_specs=[
            pl.BlockSpec((_BLK, 16), lambda i: (i, 0)),
            pl.BlockSpec((_BLK, 8), lambda i: (i, 0)),
            pl.BlockSpec((_BLK, 8), lambda i: (i, 0)),
        ],
        out_shape=[
            jax.ShapeDtypeStruct((N, 16), _F32),
            jax.ShapeDtypeStruct((N, 8), _F32),
            jax.ShapeDtypeStruct((N, 8), _F32),
        ],
    )(acc_p, r1, b1row, wg2, wr2)


def _final_body(acc2_ref, r2_ref, di_ref, b2_ref, o_ref):
    a2 = acc2_ref[0] + acc2_ref[1]
    s = a2[:, 0:8] + a2[:, 8:16]
    s = s * di_ref[...] + r2_ref[...] + b2_ref[...]
    lane = lax.broadcasted_iota(jnp.int32, s.shape, 1)
    mask = lane < 7
    sm = jnp.where(mask, s, -1e30)
    m = jnp.max(sm, axis=1, keepdims=True)
    e = jnp.where(mask, jnp.exp(sm - m), 0.0)
    lse = jnp.log(jnp.sum(e, axis=1, keepdims=True))
    o_ref[...] = s - m - lse


def _tc_final(acc2_p, r2, dinv, b2row):
    return pl.pallas_call(
        _final_body,
        grid=(N // _BLK,),
        in_specs=[
            pl.BlockSpec((2, _BLK, 16), lambda i: (0, i, 0)),
            pl.BlockSpec((_BLK, 8), lambda i: (i, 0)),
            pl.BlockSpec((_BLK, 8), lambda i: (i, 0)),
            pl.BlockSpec((1, 8), lambda i: (0, 0)),
        ],
        out_specs=pl.BlockSpec((_BLK, 8), lambda i: (i, 0)),
        out_shape=jax.ShapeDtypeStruct((N, 8), _F32),
    )(acc2_p, r2, dinv, b2row)


_ZERO_CHUNKS = ((0, 128), (128, 128), (256, 128), (384, 128), (512, 128))


def _sc_pass1(g1, src, dst, u):
    mesh = plsc.VectorSubcoreMesh(core_axis_name="c", subcore_axis_name="s")

    @functools.partial(
        pl.kernel,
        out_type=jax.ShapeDtypeStruct((NC, N_PAD, 32), _F32),
        mesh=mesh,
        compiler_params=pltpu.CompilerParams(use_tc_tiling_on_sc=False),
        scratch_types=[
            pltpu.VMEM_SHARED((N_PAD, 32), _F32),
            pltpu.VMEM((1, G), jnp.int32),
            pltpu.VMEM((1, G), jnp.int32),
            pltpu.VMEM((G,), _F32),
            pltpu.VMEM((G, 32), _F32),
            pltpu.VMEM((G, 32), _F32),
            pltpu.VMEM((G, 32), _F32),
            pltpu.SemaphoreType.DMA,
        ],
    )
    def k(g1_hbm, src_hbm, dst_hbm, u_hbm, out_hbm,
          acc, srcv, dstv, ubuf, gbuf, msgbuf, zbuf, sem):
        c = lax.axis_index("c")
        s = lax.axis_index("s")
        w = s * NC + c
        z16 = jnp.zeros((16,), _F32)
        lanes = lax.iota(jnp.int32, 16)
        e0 = jnp.where(lanes == 0, 1.0, 0.0)

        def init_body(i, carry):
            zbuf[i, pl.ds(0, 16)] = z16
            zbuf[i, pl.ds(16, 16)] = z16
            msgbuf[i, pl.ds(16, 16)] = e0
            return carry

        lax.fori_loop(0, G, init_body, 0)

        base = s * RPS
        for off, cnt in _ZERO_CHUNKS:
            pltpu.sync_copy(zbuf.at[pl.ds(0, cnt)], acc.at[pl.ds(base + off, cnt)])
        plsc.subcore_barrier()

        n_t = 78 + jnp.where(w < N_GROUPS - 78 * NW, 1, 0)

        def group_body(t, carry):
            off_e = (w + NW * t) * G
            pltpu.sync_copy(src_hbm.at[pl.ds(off_e, G)], srcv.at[0])
            pltpu.sync_copy(dst_hbm.at[pl.ds(off_e, G)], dstv.at[0])
            pltpu.sync_copy(u_hbm.at[pl.ds(off_e, G)], ubuf)
            pltpu.async_copy(g1_hbm.at[srcv.at[0]], gbuf, sem).wait()

            def edge_body(j, cc):
                uvec = ubuf[pl.ds(j * 16, 16)]
                for kk in range(16):
                    i = j * 16 + kk
                    us = uvec[kk]
                    g0 = gbuf[i, pl.ds(0, 16)]
                    gd = gbuf[i, pl.ds(16, 16)]
                    msgbuf[i, pl.ds(0, 16)] = g0 + us * gd
                return cc

            lax.fori_loop(0, G // 16, edge_body, 0)
            pltpu.sync_copy(msgbuf, acc.at[dstv.at[0]], add=True)
            return carry

        lax.fori_loop(0, n_t, group_body, 0)
        plsc.subcore_barrier()

        for off, cnt in _ZERO_CHUNKS:
            pltpu.sync_copy(acc.at[pl.ds(base + off, cnt)],
                            out_hbm.at[c, pl.ds(base + off, cnt)])

    return k(g1, src, dst, u)


def _sc_pass2(g2, src, dst, u):
    mesh = plsc.VectorSubcoreMesh(core_axis_name="c", subcore_axis_name="s")

    @functools.partial(
        pl.kernel,
        out_type=jax.ShapeDtypeStruct((NC, N_PAD, 16), _F32),
        mesh=mesh,
        compiler_params=pltpu.CompilerParams(use_tc_tiling_on_sc=False),
        scratch_types=[
            pltpu.VMEM_SHARED((N_PAD, 16), _F32),
            pltpu.VMEM((1, G), jnp.int32),
            pltpu.VMEM((1, G), jnp.int32),
            pltpu.VMEM((G,), _F32),
            pltpu.VMEM((G, 16), _F32),
            pltpu.VMEM((G, 16), _F32),
            pltpu.VMEM((G, 16), _F32),
            pltpu.SemaphoreType.DMA,
        ],
    )
    def k(g2_hbm, src_hbm, dst_hbm, u_hbm, out_hbm,
          acc, srcv, dstv, ubuf, gbuf, msgbuf, zbuf, sem):
        c = lax.axis_index("c")
        s = lax.axis_index("s")
        w = s * NC + c
        z16 = jnp.zeros((16,), _F32)
        lanes = lax.iota(jnp.int32, 16)
        lo_mask = lanes < 8

        def init_body(i, carry):
            zbuf[i, pl.ds(0, 16)] = z16
            return carry

        lax.fori_loop(0, G, init_body, 0)

        base = s * RPS
        for off, cnt in _ZERO_CHUNKS:
            pltpu.sync_copy(zbuf.at[pl.ds(0, cnt)], acc.at[pl.ds(base + off, cnt)])
        plsc.subcore_barrier()

        n_t = 78 + jnp.where(w < N_GROUPS - 78 * NW, 1, 0)

        def group_body(t, carry):
            off_e = (w + NW * t) * G
            pltpu.sync_copy(src_hbm.at[pl.ds(off_e, G)], srcv.at[0])
            pltpu.sync_copy(dst_hbm.at[pl.ds(off_e, G)], dstv.at[0])
            pltpu.sync_copy(u_hbm.at[pl.ds(off_e, G)], ubuf)
            pltpu.async_copy(g2_hbm.at[srcv.at[0]], gbuf, sem).wait()

            def edge_body(j, cc):
                uvec = ubuf[pl.ds(j * 16, 16)]
                for kk in range(16):
                    i = j * 16 + kk
                    us = uvec[kk]
                    row = gbuf[i, pl.ds(0, 16)]
                    wv = jnp.where(lo_mask, 1.0, us)
                    msgbuf[i, pl.ds(0, 16)] = row * wv
                return cc

            lax.fori_loop(0, G // 16, edge_body, 0)
            pltpu.sync_copy(msgbuf, acc.at[dstv.at[0]], add=True)
            return carry

        lax.fori_loop(0, n_t, group_body, 0)
        plsc.subcore_barrier()

        for off, cnt in _ZERO_CHUNKS:
            pltpu.sync_copy(acc.at[pl.ds(base + off, cnt)],
                            out_hbm.at[c, pl.ds(base + off, cnt)])

    return k(g2, src, dst, u)


def kernel(x, edge_index, edge_attr, W1, root1, b1, W2, root2, b2):
    src = edge_index[0].astype(jnp.int32)
    dst = edge_index[1].astype(jnp.int32)
    u = edge_attr[:, 0].astype(_F32)

    wcat1 = jnp.concatenate([W1[0], W1[1] - W1[0], root1], axis=1)
    g1, r1 = _tc_prep1(x, wcat1)
    acc_p = _sc_pass1(g1, src, dst, u)

    wg2 = jnp.zeros((16, 16), _F32)
    wg2 = wg2.at[:, 0:7].set(W2[0]).at[:, 8:15].set(W2[1] - W2[0])
    wr2 = jnp.zeros((16, 8), _F32).at[:, 0:7].set(root2)
    b1row = b1.reshape(1, 16).astype(_F32)
    g2, r2, dinv = _tc_mid(acc_p, r1, b1row, wg2, wr2)

    acc2_p = _sc_pass2(g2, src, dst, u)
    b2row = jnp.zeros((1, 8), _F32).at[0, 0:7].set(b2)
    out8 = _tc_final(acc2_p, r2, dinv, b2row)
    return out8[:, 0:7]


# speedup vs baseline: 10.3684x; 1.2845x over previous
import functools

import jax
import jax.numpy as jnp
from jax import lax
from jax.experimental import pallas as pl
from jax.experimental.pallas import tpu as pltpu
from jax.experimental.pallas import tpu_sc as plsc

N = 10000
E = 320000
G = 128
N_GROUPS = E // G
NC = 2
NS = 16
NW = NC * NS
N_PAD = 10240
RPS = N_PAD // NS

_F32 = jnp.float32


_BLK = 1000


def _prep1_body(x_ref, w_ref, g1_ref, r1_ref):
    y = jnp.dot(x_ref[...], w_ref[...], preferred_element_type=_F32)
    g1_ref[...] = y[:, 0:32]
    r1_ref[...] = y[:, 32:48]


def _tc_prep1(x, wcat1):
    return pl.pallas_call(
        _prep1_body,
        grid=(N // _BLK,),
        in_specs=[
            pl.BlockSpec((_BLK, 128), lambda i: (i, 0)),
            pl.BlockSpec((128, 48), lambda i: (0, 0)),
        ],
        out_specs=[
            pl.BlockSpec((_BLK, 32), lambda i: (i, 0)),
            pl.BlockSpec((_BLK, 16), lambda i: (i, 0)),
        ],
        out_shape=[
            jax.ShapeDtypeStruct((N, 32), _F32),
            jax.ShapeDtypeStruct((N, 16), _F32),
        ],
    )(x, wcat1)


def _mid_body(accp_ref, r1_ref, b1_ref, wg2_ref, wr2_ref, g2_ref, r2_ref, di_ref):
    a = accp_ref[0] + accp_ref[1]
    agg = a[:, 0:16]
    deg = jnp.maximum(a[:, 16:17], 1.0)
    v = agg / deg + r1_ref[...] + b1_ref[...]
    h = jnp.where(v > 0, v, jnp.exp(v) - 1.0)
    g2_ref[...] = jnp.dot(h, wg2_ref[...], preferred_element_type=_F32)
    r2_ref[...] = jnp.dot(h, wr2_ref[...], preferred_element_type=_F32)
    di_ref[...] = jnp.broadcast_to(1.0 / deg, deg.shape[:1] + (8,))


def _tc_mid(acc_p, r1, b1row, wg2, wr2):
    return pl.pallas_call(
        _mid_body,
        grid=(N // _BLK,),
        in_specs=[
            pl.BlockSpec((2, _BLK, 32), lambda i: (0, i, 0)),
            pl.BlockSpec((_BLK, 16), lambda i: (i, 0)),
            pl.BlockSpec((1, 16), lambda i: (0, 0)),
            pl.BlockSpec((16, 16), lambda i: (0, 0)),
            pl.BlockSpec((16, 8), lambda i: (0, 0)),
        ],
        out_specs=[
            pl.BlockSpec((_BLK, 16), lambda i: (i, 0)),
            pl.BlockSpec((_BLK, 8), lambda i: (i, 0)),
            pl.BlockSpec((_BLK, 8), lambda i: (i, 0)),
        ],
        out_shape=[
            jax.ShapeDtypeStruct((N, 16), _F32),
            jax.ShapeDtypeStruct((N, 8), _F32),
            jax.ShapeDtypeStruct((N, 8), _F32),
        ],
    )(acc_p, r1, b1row, wg2, wr2)


def _final_body(acc2_ref, r2_ref, di_ref, b2_ref, o_ref):
    a2 = acc2_ref[0] + acc2_ref[1]
    s = a2[:, 0:8] + a2[:, 8:16]
    s = s * di_ref[...] + r2_ref[...] + b2_ref[...]
    lane = lax.broadcasted_iota(jnp.int32, s.shape, 1)
    mask = lane < 7
    sm = jnp.where(mask, s, -1e30)
    m = jnp.max(sm, axis=1, keepdims=True)
    e = jnp.where(mask, jnp.exp(sm - m), 0.0)
    lse = jnp.log(jnp.sum(e, axis=1, keepdims=True))
    o_ref[...] = s - m - lse


def _tc_final(acc2_p, r2, dinv, b2row):
    return pl.pallas_call(
        _final_body,
        grid=(N // _BLK,),
        in_specs=[
            pl.BlockSpec((2, _BLK, 16), lambda i: (0, i, 0)),
            pl.BlockSpec((_BLK, 8), lambda i: (i, 0)),
            pl.BlockSpec((_BLK, 8), lambda i: (i, 0)),
            pl.BlockSpec((1, 8), lambda i: (0, 0)),
        ],
        out_specs=pl.BlockSpec((_BLK, 8), lambda i: (i, 0)),
        out_shape=jax.ShapeDtypeStruct((N, 8), _F32),
    )(acc2_p, r2, dinv, b2row)


_ZERO_CHUNKS = ((0, 128), (128, 128), (256, 128), (384, 128), (512, 128))


def _sc_pass1(g1, src, dst, u):
    mesh = plsc.VectorSubcoreMesh(core_axis_name="c", subcore_axis_name="s")

    @functools.partial(
        pl.kernel,
        out_type=jax.ShapeDtypeStruct((NC, N_PAD, 32), _F32),
        mesh=mesh,
        compiler_params=pltpu.CompilerParams(use_tc_tiling_on_sc=False),
        scratch_types=[
            pltpu.VMEM_SHARED((N_PAD, 32), _F32),
            pltpu.VMEM((1, G), jnp.int32),
            pltpu.VMEM((1, G), jnp.int32),
            pltpu.VMEM((G,), _F32),
            pltpu.VMEM((G, 32), _F32),
            pltpu.VMEM((G, 32), _F32),
            pltpu.VMEM((1, G), jnp.int32),
            pltpu.VMEM((1, G), jnp.int32),
            pltpu.VMEM((G,), _F32),
            pltpu.VMEM((G, 32), _F32),
            pltpu.VMEM((G, 32), _F32),
            pltpu.VMEM((G, 32), _F32),
            pltpu.SemaphoreType.DMA,
            pltpu.SemaphoreType.DMA,
        ],
    )
    def k(g1_hbm, src_hbm, dst_hbm, u_hbm, out_hbm,
          acc, srcv, dstv, ubuf, gbuf, msgbuf,
          srcv2, dstv2, ubuf2, gbuf2, msgbuf2, zbuf, sem, sem2):
        c = lax.axis_index("c")
        s = lax.axis_index("s")
        w = s * NC + c
        z16 = jnp.zeros((16,), _F32)
        lanes = lax.iota(jnp.int32, 16)
        e0 = jnp.where(lanes == 0, 1.0, 0.0)

        def init_body(i, carry):
            zbuf[i, pl.ds(0, 16)] = z16
            zbuf[i, pl.ds(16, 16)] = z16
            msgbuf[i, pl.ds(16, 16)] = e0
            msgbuf2[i, pl.ds(16, 16)] = e0
            return carry

        lax.fori_loop(0, G, init_body, 0)

        base = s * RPS
        for off, cnt in _ZERO_CHUNKS:
            pltpu.sync_copy(zbuf.at[pl.ds(0, cnt)], acc.at[pl.ds(base + off, cnt)])
        plsc.subcore_barrier()

        def make_edge_body(ub, gb, mb):
            def edge_body(j, cc):
                uvec = ub[pl.ds(j * 16, 16)]
                for kk in range(16):
                    i = j * 16 + kk
                    us = uvec[kk]
                    g0 = gb[i, pl.ds(0, 16)]
                    gd = gb[i, pl.ds(16, 16)]
                    mb[i, pl.ds(0, 16)] = g0 + us * gd
                return cc
            return edge_body

        def pair_body(t2, carry):
            off_a = (w + NW * (2 * t2)) * G
            off_b = (w + NW * (2 * t2 + 1)) * G
            pltpu.sync_copy(src_hbm.at[pl.ds(off_a, G)], srcv.at[0])
            da = pltpu.async_copy(g1_hbm.at[srcv.at[0]], gbuf, sem)
            pltpu.sync_copy(src_hbm.at[pl.ds(off_b, G)], srcv2.at[0])
            db = pltpu.async_copy(g1_hbm.at[srcv2.at[0]], gbuf2, sem2)
            pltpu.sync_copy(dst_hbm.at[pl.ds(off_a, G)], dstv.at[0])
            pltpu.sync_copy(u_hbm.at[pl.ds(off_a, G)], ubuf)
            pltpu.sync_copy(dst_hbm.at[pl.ds(off_b, G)], dstv2.at[0])
            pltpu.sync_copy(u_hbm.at[pl.ds(off_b, G)], ubuf2)
            da.wait()
            lax.fori_loop(0, G // 16, make_edge_body(ubuf, gbuf, msgbuf), 0)
            pltpu.sync_copy(msgbuf, acc.at[dstv.at[0]], add=True)
            db.wait()
            lax.fori_loop(0, G // 16, make_edge_body(ubuf2, gbuf2, msgbuf2), 0)
            pltpu.sync_copy(msgbuf2, acc.at[dstv2.at[0]], add=True)
            return carry

        lax.fori_loop(0, 39, pair_body, 0)

        @pl.when(w < N_GROUPS - 78 * NW)
        def _tail():
            off_e = (w + NW * 78) * G
            pltpu.sync_copy(src_hbm.at[pl.ds(off_e, G)], srcv.at[0])
            pltpu.sync_copy(dst_hbm.at[pl.ds(off_e, G)], dstv.at[0])
            pltpu.sync_copy(u_hbm.at[pl.ds(off_e, G)], ubuf)
            pltpu.async_copy(g1_hbm.at[srcv.at[0]], gbuf, sem).wait()
            lax.fori_loop(0, G // 16, make_edge_body(ubuf, gbuf, msgbuf), 0)
            pltpu.sync_copy(msgbuf, acc.at[dstv.at[0]], add=True)
        plsc.subcore_barrier()

        for off, cnt in _ZERO_CHUNKS:
            pltpu.sync_copy(acc.at[pl.ds(base + off, cnt)],
                            out_hbm.at[c, pl.ds(base + off, cnt)])

    return k(g1, src, dst, u)


def _sc_pass2(g2, src, dst, u):
    mesh = plsc.VectorSubcoreMesh(core_axis_name="c", subcore_axis_name="s")

    @functools.partial(
        pl.kernel,
        out_type=jax.ShapeDtypeStruct((NC, N_PAD, 16), _F32),
        mesh=mesh,
        compiler_params=pltpu.CompilerParams(use_tc_tiling_on_sc=False),
        scratch_types=[
            pltpu.VMEM_SHARED((N_PAD, 16), _F32),
            pltpu.VMEM((1, G), jnp.int32),
            pltpu.VMEM((1, G), jnp.int32),
            pltpu.VMEM((G,), _F32),
            pltpu.VMEM((G, 16), _F32),
            pltpu.VMEM((G, 16), _F32),
            pltpu.VMEM((1, G), jnp.int32),
            pltpu.VMEM((1, G), jnp.int32),
            pltpu.VMEM((G,), _F32),
            pltpu.VMEM((G, 16), _F32),
            pltpu.VMEM((G, 16), _F32),
            pltpu.VMEM((G, 16), _F32),
            pltpu.SemaphoreType.DMA,
            pltpu.SemaphoreType.DMA,
        ],
    )
    def k(g2_hbm, src_hbm, dst_hbm, u_hbm, out_hbm,
          acc, srcv, dstv, ubuf, gbuf, msgbuf,
          srcv2, dstv2, ubuf2, gbuf2, msgbuf2, zbuf, sem, sem2):
        c = lax.axis_index("c")
        s = lax.axis_index("s")
        w = s * NC + c
        z16 = jnp.zeros((16,), _F32)
        lanes = lax.iota(jnp.int32, 16)
        lo_mask = lanes < 8

        def init_body(i, carry):
            zbuf[i, pl.ds(0, 16)] = z16
            return carry

        lax.fori_loop(0, G, init_body, 0)

        base = s * RPS
        for off, cnt in _ZERO_CHUNKS:
            pltpu.sync_copy(zbuf.at[pl.ds(0, cnt)], acc.at[pl.ds(base + off, cnt)])
        plsc.subcore_barrier()

        def make_edge_body(ub, gb, mb):
            def edge_body(j, cc):
                uvec = ub[pl.ds(j * 16, 16)]
                for kk in range(16):
                    i = j * 16 + kk
                    us = uvec[kk]
                    row = gb[i, pl.ds(0, 16)]
                    wv = jnp.where(lo_mask, 1.0, us)
                    mb[i, pl.ds(0, 16)] = row * wv
                return cc
            return edge_body

        def pair_body(t2, carry):
            off_a = (w + NW * (2 * t2)) * G
            off_b = (w + NW * (2 * t2 + 1)) * G
            pltpu.sync_copy(src_hbm.at[pl.ds(off_a, G)], srcv.at[0])
            da = pltpu.async_copy(g2_hbm.at[srcv.at[0]], gbuf, sem)
            pltpu.sync_copy(src_hbm.at[pl.ds(off_b, G)], srcv2.at[0])
            db = pltpu.async_copy(g2_hbm.at[srcv2.at[0]], gbuf2, sem2)
            pltpu.sync_copy(dst_hbm.at[pl.ds(off_a, G)], dstv.at[0])
            pltpu.sync_copy(u_hbm.at[pl.ds(off_a, G)], ubuf)
            pltpu.sync_copy(dst_hbm.at[pl.ds(off_b, G)], dstv2.at[0])
            pltpu.sync_copy(u_hbm.at[pl.ds(off_b, G)], ubuf2)
            da.wait()
            lax.fori_loop(0, G // 16, make_edge_body(ubuf, gbuf, msgbuf), 0)
            pltpu.sync_copy(msgbuf, acc.at[dstv.at[0]], add=True)
            db.wait()
            lax.fori_loop(0, G // 16, make_edge_body(ubuf2, gbuf2, msgbuf2), 0)
            pltpu.sync_copy(msgbuf2, acc.at[dstv2.at[0]], add=True)
            return carry

        lax.fori_loop(0, 39, pair_body, 0)

        @pl.when(w < N_GROUPS - 78 * NW)
        def _tail():
            off_e = (w + NW * 78) * G
            pltpu.sync_copy(src_hbm.at[pl.ds(off_e, G)], srcv.at[0])
            pltpu.sync_copy(dst_hbm.at[pl.ds(off_e, G)], dstv.at[0])
            pltpu.sync_copy(u_hbm.at[pl.ds(off_e, G)], ubuf)
            pltpu.async_copy(g2_hbm.at[srcv.at[0]], gbuf, sem).wait()
            lax.fori_loop(0, G // 16, make_edge_body(ubuf, gbuf, msgbuf), 0)
            pltpu.sync_copy(msgbuf, acc.at[dstv.at[0]], add=True)
        plsc.subcore_barrier()

        for off, cnt in _ZERO_CHUNKS:
            pltpu.sync_copy(acc.at[pl.ds(base + off, cnt)],
                            out_hbm.at[c, pl.ds(base + off, cnt)])

    return k(g2, src, dst, u)


def kernel(x, edge_index, edge_attr, W1, root1, b1, W2, root2, b2):
    src = edge_index[0].astype(jnp.int32)
    dst = edge_index[1].astype(jnp.int32)
    u = edge_attr[:, 0].astype(_F32)

    wcat1 = jnp.concatenate([W1[0], W1[1] - W1[0], root1], axis=1)
    g1, r1 = _tc_prep1(x, wcat1)
    acc_p = _sc_pass1(g1, src, dst, u)

    wg2 = jnp.zeros((16, 16), _F32)
    wg2 = wg2.at[:, 0:7].set(W2[0]).at[:, 8:15].set(W2[1] - W2[0])
    wr2 = jnp.zeros((16, 8), _F32).at[:, 0:7].set(root2)
    b1row = b1.reshape(1, 16).astype(_F32)
    g2, r2, dinv = _tc_mid(acc_p, r1, b1row, wg2, wr2)

    acc2_p = _sc_pass2(g2, src, dst, u)
    b2row = jnp.zeros((1, 8), _F32).at[0, 0:7].set(b2)
    out8 = _tc_final(acc2_p, r2, dinv, b2row)
    return out8[:, 0:7]


# speedup vs baseline: 10.8670x; 1.0481x over previous
import functools

import jax
import jax.numpy as jnp
from jax import lax
from jax.experimental import pallas as pl
from jax.experimental.pallas import tpu as pltpu
from jax.experimental.pallas import tpu_sc as plsc

N = 10000
E = 320000
G = 128
N_GROUPS = E // G
NC = 2
NS = 16
NW = NC * NS
N_PAD = 10240
RPS = N_PAD // NS

_F32 = jnp.float32


_BLK = 1000


def _prep1_body(x_ref, w_ref, g1_ref, r1_ref):
    y = jnp.dot(x_ref[...], w_ref[...], preferred_element_type=_F32)
    g1_ref[...] = y[:, 0:32]
    r1_ref[...] = y[:, 32:48]


def _tc_prep1(x, wcat1):
    return pl.pallas_call(
        _prep1_body,
        grid=(N // _BLK,),
        in_specs=[
            pl.BlockSpec((_BLK, 128), lambda i: (i, 0)),
            pl.BlockSpec((128, 48), lambda i: (0, 0)),
        ],
        out_specs=[
            pl.BlockSpec((_BLK, 32), lambda i: (i, 0)),
            pl.BlockSpec((_BLK, 16), lambda i: (i, 0)),
        ],
        out_shape=[
            jax.ShapeDtypeStruct((N, 32), _F32),
            jax.ShapeDtypeStruct((N, 16), _F32),
        ],
    )(x, wcat1)


def _mid_body(accp_ref, r1_ref, b1_ref, wg2_ref, wr2_ref, g2_ref, r2_ref, di_ref):
    a = accp_ref[0] + accp_ref[1]
    agg = a[:, 0:16]
    deg = jnp.maximum(a[:, 16:17], 1.0)
    v = agg / deg + r1_ref[...] + b1_ref[...]
    h = jnp.where(v > 0, v, jnp.exp(v) - 1.0)
    g2_ref[...] = jnp.dot(h, wg2_ref[...], preferred_element_type=_F32)
    r2_ref[...] = jnp.dot(h, wr2_ref[...], preferred_element_type=_F32)
    di_ref[...] = jnp.broadcast_to(1.0 / deg, deg.shape[:1] + (8,))


def _tc_mid(acc_p, r1, b1row, wg2, wr2):
    return pl.pallas_call(
        _mid_body,
        grid=(N // _BLK,),
        in_specs=[
            pl.BlockSpec((2, _BLK, 32), lambda i: (0, i, 0)),
            pl.BlockSpec((_BLK, 16), lambda i: (i, 0)),
            pl.BlockSpec((1, 16), lambda i: (0, 0)),
            pl.BlockSpec((16, 16), lambda i: (0, 0)),
            pl.BlockSpec((16, 8), lambda i: (0, 0)),
        ],
        out_specs=[
            pl.BlockSpec((_BLK, 16), lambda i: (i, 0)),
            pl.BlockSpec((_BLK, 8), lambda i: (i, 0)),
            pl.BlockSpec((_BLK, 8), lambda i: (i, 0)),
        ],
        out_shape=[
            jax.ShapeDtypeStruct((N, 16), _F32),
            jax.ShapeDtypeStruct((N, 8), _F32),
            jax.ShapeDtypeStruct((N, 8), _F32),
        ],
    )(acc_p, r1, b1row, wg2, wr2)


def _final_body(acc2_ref, r2_ref, di_ref, b2_ref, o_ref):
    a2 = acc2_ref[0] + acc2_ref[1]
    s = a2[:, 0:8] + a2[:, 8:16]
    s = s * di_ref[...] + r2_ref[...] + b2_ref[...]
    lane = lax.broadcasted_iota(jnp.int32, s.shape, 1)
    mask = lane < 7
    sm = jnp.where(mask, s, -1e30)
    m = jnp.max(sm, axis=1, keepdims=True)
    e = jnp.where(mask, jnp.exp(sm - m), 0.0)
    lse = jnp.log(jnp.sum(e, axis=1, keepdims=True))
    o_ref[...] = s - m - lse


def _tc_final(acc2_p, r2, dinv, b2row):
    return pl.pallas_call(
        _final_body,
        grid=(N // _BLK,),
        in_specs=[
            pl.BlockSpec((2, _BLK, 16), lambda i: (0, i, 0)),
            pl.BlockSpec((_BLK, 8), lambda i: (i, 0)),
            pl.BlockSpec((_BLK, 8), lambda i: (i, 0)),
            pl.BlockSpec((1, 8), lambda i: (0, 0)),
        ],
        out_specs=pl.BlockSpec((_BLK, 8), lambda i: (i, 0)),
        out_shape=jax.ShapeDtypeStruct((N, 8), _F32),
    )(acc2_p, r2, dinv, b2row)


_ZERO_CHUNKS = ((0, 128), (128, 128), (256, 128), (384, 128), (512, 128))

NSETS = 4
NPAIRS = 19


def _sc_edge_pass(tbl, src, dst, u, width, layer1):
    mesh = plsc.VectorSubcoreMesh(core_axis_name="c", subcore_axis_name="s")
    scratch = [pltpu.VMEM_SHARED((N_PAD, width), _F32)]
    for _ in range(NSETS):
        scratch += [
            pltpu.VMEM((1, G), jnp.int32),
            pltpu.VMEM((1, G), jnp.int32),
            pltpu.VMEM((G,), _F32),
            pltpu.VMEM((G, width), _F32),
            pltpu.VMEM((G, width), _F32),
        ]
    scratch.append(pltpu.VMEM((G, width), _F32))
    scratch += [pltpu.SemaphoreType.DMA] * (2 * NSETS)

    @functools.partial(
        pl.kernel,
        out_type=jax.ShapeDtypeStruct((NC, N_PAD, width), _F32),
        mesh=mesh,
        compiler_params=pltpu.CompilerParams(use_tc_tiling_on_sc=False),
        scratch_types=scratch,
    )
    def k(tbl_hbm, src_hbm, dst_hbm, u_hbm, out_hbm, acc, *rest):
        sets = []
        for i in range(NSETS):
            sets.append(rest[5 * i:5 * i + 5])
        zbuf = rest[5 * NSETS]
        semg = rest[5 * NSETS + 1:5 * NSETS + 1 + NSETS]
        sems = rest[5 * NSETS + 1 + NSETS:]

        c = lax.axis_index("c")
        s = lax.axis_index("s")
        w = s * NC + c
        z16 = jnp.zeros((16,), _F32)
        lanes = lax.iota(jnp.int32, 16)
        e0 = jnp.where(lanes == 0, 1.0, 0.0)
        lo_mask = lanes < 8

        def init_body(i, carry):
            for half in range(width // 16):
                zbuf[i, pl.ds(16 * half, 16)] = z16
            if layer1:
                for st in sets:
                    st[4][i, pl.ds(16, 16)] = e0
            return carry

        lax.fori_loop(0, G, init_body, 0)

        base = s * RPS
        for off, cnt in _ZERO_CHUNKS:
            pltpu.sync_copy(zbuf.at[pl.ds(0, cnt)], acc.at[pl.ds(base + off, cnt)])
        plsc.subcore_barrier()

        def make_edge_body(ub, gb, mb):
            def edge_body(j, cc):
                uvec = ub[pl.ds(j * 16, 16)]
                for kk in range(16):
                    i = j * 16 + kk
                    us = uvec[kk]
                    if layer1:
                        g0 = gb[i, pl.ds(0, 16)]
                        gd = gb[i, pl.ds(16, 16)]
                        mb[i, pl.ds(0, 16)] = g0 + us * gd
                    else:
                        row = gb[i, pl.ds(0, 16)]
                        wv = jnp.where(lo_mask, 1.0, us)
                        mb[i, pl.ds(0, 16)] = row * wv
                return cc
            return edge_body

        def run_group(off_e, st, gsem, ssem, sync_scatter):
            srcv, dstv, ubuf, gbuf, msgbuf = st
            pltpu.sync_copy(src_hbm.at[pl.ds(off_e, G)], srcv.at[0])
            dg = pltpu.async_copy(tbl_hbm.at[srcv.at[0]], gbuf, gsem)
            pltpu.sync_copy(dst_hbm.at[pl.ds(off_e, G)], dstv.at[0])
            pltpu.sync_copy(u_hbm.at[pl.ds(off_e, G)], ubuf)
            dg.wait()
            lax.fori_loop(0, G // 16, make_edge_body(ubuf, gbuf, msgbuf), 0)
            if sync_scatter:
                pltpu.sync_copy(msgbuf, acc.at[dstv.at[0]], add=True)
                return None
            return pltpu.async_copy(msgbuf, acc.at[dstv.at[0]], ssem, add=True)

        def iter_body(t, carry):
            g0 = w + NW * (NSETS * t)
            dgs = []
            for i in range(NSETS):
                st = sets[i]
                pltpu.sync_copy(src_hbm.at[pl.ds((g0 + NW * i) * G, G)],
                                st[0].at[0])
                dgs.append(pltpu.async_copy(tbl_hbm.at[st[0].at[0]], st[3],
                                            semg[i]))
            for i in range(NSETS):
                st = sets[i]
                pltpu.sync_copy(dst_hbm.at[pl.ds((g0 + NW * i) * G, G)],
                                st[1].at[0])
                pltpu.sync_copy(u_hbm.at[pl.ds((g0 + NW * i) * G, G)], st[2])
            dss = []
            for i in range(NSETS):
                st = sets[i]
                dgs[i].wait()
                lax.fori_loop(0, G // 16, make_edge_body(st[2], st[3], st[4]), 0)
                dss.append(pltpu.async_copy(st[4], acc.at[st[1].at[0]],
                                            sems[i], add=True))
            for d in dss:
                d.wait()
            return carry

        lax.fori_loop(0, NPAIRS, iter_body, 0)

        run_group((w + NW * 76) * G, sets[0], semg[0], sems[0], True)
        run_group((w + NW * 77) * G, sets[1], semg[1], sems[1], True)

        @pl.when(w < N_GROUPS - 78 * NW)
        def _tail():
            run_group((w + NW * 78) * G, sets[2], semg[2], sems[2], True)

        plsc.subcore_barrier()

        for off, cnt in _ZERO_CHUNKS:
            pltpu.sync_copy(acc.at[pl.ds(base + off, cnt)],
                            out_hbm.at[c, pl.ds(base + off, cnt)])

    return k(tbl, src, dst, u)


def kernel(x, edge_index, edge_attr, W1, root1, b1, W2, root2, b2):
    src = edge_index[0].astype(jnp.int32)
    dst = edge_index[1].astype(jnp.int32)
    u = edge_attr[:, 0].astype(_F32)

    wcat1 = jnp.concatenate([W1[0], W1[1] - W1[0], root1], axis=1)
    g1, r1 = _tc_prep1(x, wcat1)
    acc_p = _sc_edge_pass(g1, src, dst, u, 32, True)

    wg2 = jnp.zeros((16, 16), _F32)
    wg2 = wg2.at[:, 0:7].set(W2[0]).at[:, 8:15].set(W2[1] - W2[0])
    wr2 = jnp.zeros((16, 8), _F32).at[:, 0:7].set(root2)
    b1row = b1.reshape(1, 16).astype(_F32)
    g2, r2, dinv = _tc_mid(acc_p, r1, b1row, wg2, wr2)

    acc2_p = _sc_edge_pass(g2, src, dst, u, 16, False)
    b2row = jnp.zeros((1, 8), _F32).at[0, 0:7].set(b2)
    out8 = _tc_final(acc2_p, r2, dinv, b2row)
    return out8[:, 0:7]


# speedup vs baseline: 15.5328x; 1.4294x over previous
import functools

import jax
import jax.numpy as jnp
from jax import lax
from jax.experimental import pallas as pl
from jax.experimental.pallas import tpu as pltpu
from jax.experimental.pallas import tpu_sc as plsc

N = 10000
E = 320000
G = 128
N_GROUPS = E // G
NC = 2
NS = 16
NW = NC * NS
N_PAD = 10240
RPS = N_PAD // NS

_F32 = jnp.float32


_BLK = 1000


def _prep1_body(x_ref, w_ref, g1_ref, r1_ref):
    y = jnp.dot(x_ref[...], w_ref[...], preferred_element_type=_F32)
    g1_ref[...] = y[:, 0:32]
    r1_ref[...] = y[:, 32:48]


def _tc_prep1(x, wcat1):
    return pl.pallas_call(
        _prep1_body,
        grid=(N // _BLK,),
        in_specs=[
            pl.BlockSpec((_BLK, 128), lambda i: (i, 0)),
            pl.BlockSpec((128, 48), lambda i: (0, 0)),
        ],
        out_specs=[
            pl.BlockSpec((_BLK, 32), lambda i: (i, 0)),
            pl.BlockSpec((_BLK, 16), lambda i: (i, 0)),
        ],
        out_shape=[
            jax.ShapeDtypeStruct((N, 32), _F32),
            jax.ShapeDtypeStruct((N, 16), _F32),
        ],
    )(x, wcat1)


def _mid_body(accp_ref, r1_ref, b1_ref, wg2_ref, wr2_ref, g2_ref, r2_ref, di_ref):
    a = accp_ref[0] + accp_ref[1]
    agg = a[:, 0:16]
    deg = jnp.maximum(a[:, 16:17], 1.0)
    v = agg / deg + r1_ref[...] + b1_ref[...]
    h = jnp.where(v > 0, v, jnp.exp(v) - 1.0)
    g2_ref[...] = jnp.dot(h, wg2_ref[...], preferred_element_type=_F32)
    r2_ref[...] = jnp.dot(h, wr2_ref[...], preferred_element_type=_F32)
    di_ref[...] = jnp.broadcast_to(1.0 / deg, deg.shape[:1] + (8,))


def _tc_mid(acc_p, r1, b1row, wg2, wr2):
    return pl.pallas_call(
        _mid_body,
        grid=(N // _BLK,),
        in_specs=[
            pl.BlockSpec((2, _BLK, 32), lambda i: (0, i, 0)),
            pl.BlockSpec((_BLK, 16), lambda i: (i, 0)),
            pl.BlockSpec((1, 16), lambda i: (0, 0)),
            pl.BlockSpec((16, 16), lambda i: (0, 0)),
            pl.BlockSpec((16, 8), lambda i: (0, 0)),
        ],
        out_specs=[
            pl.BlockSpec((_BLK, 16), lambda i: (i, 0)),
            pl.BlockSpec((_BLK, 8), lambda i: (i, 0)),
            pl.BlockSpec((_BLK, 8), lambda i: (i, 0)),
        ],
        out_shape=[
            jax.ShapeDtypeStruct((N, 16), _F32),
            jax.ShapeDtypeStruct((N, 8), _F32),
            jax.ShapeDtypeStruct((N, 8), _F32),
        ],
    )(acc_p, r1, b1row, wg2, wr2)


def _final_body(acc2_ref, r2_ref, di_ref, b2_ref, o_ref):
    a2 = acc2_ref[0] + acc2_ref[1]
    s = a2[:, 0:8] + a2[:, 8:16]
    s = s * di_ref[...] + r2_ref[...] + b2_ref[...]
    lane = lax.broadcasted_iota(jnp.int32, s.shape, 1)
    mask = lane < 7
    sm = jnp.where(mask, s, -1e30)
    m = jnp.max(sm, axis=1, keepdims=True)
    e = jnp.where(mask, jnp.exp(sm - m), 0.0)
    lse = jnp.log(jnp.sum(e, axis=1, keepdims=True))
    o_ref[...] = s - m - lse


def _tc_final(acc2_p, r2, dinv, b2row):
    return pl.pallas_call(
        _final_body,
        grid=(N // _BLK,),
        in_specs=[
            pl.BlockSpec((2, _BLK, 16), lambda i: (0, i, 0)),
            pl.BlockSpec((_BLK, 8), lambda i: (i, 0)),
            pl.BlockSpec((_BLK, 8), lambda i: (i, 0)),
            pl.BlockSpec((1, 8), lambda i: (0, 0)),
        ],
        out_specs=pl.BlockSpec((_BLK, 8), lambda i: (i, 0)),
        out_shape=jax.ShapeDtypeStruct((N, 8), _F32),
    )(acc2_p, r2, dinv, b2row)


_ZERO_CHUNKS = ((0, 128), (128, 128), (256, 128), (384, 128), (512, 128))

NSETS = 4
NPAIRS = 19


def _sc_edge_pass(tbl, src, dst, u, width, layer1):
    mesh = plsc.VectorSubcoreMesh(core_axis_name="c", subcore_axis_name="s")
    scratch = [
        pltpu.VMEM_SHARED((N_PAD, width), _F32),
        pltpu.VMEM((NSETS, G), jnp.int32),
        pltpu.VMEM((NSETS, G), jnp.int32),
        pltpu.VMEM((NSETS * G,), _F32),
    ]
    for _ in range(NSETS):
        scratch += [
            pltpu.VMEM((G, width), _F32),
            pltpu.VMEM((G, width), _F32),
        ]
    scratch.append(pltpu.VMEM((G, width), _F32))
    scratch += [pltpu.SemaphoreType.DMA] * (2 * NSETS)

    @functools.partial(
        pl.kernel,
        out_type=jax.ShapeDtypeStruct((NC, N_PAD, width), _F32),
        mesh=mesh,
        compiler_params=pltpu.CompilerParams(use_tc_tiling_on_sc=False),
        scratch_types=scratch,
    )
    def k(tbl_hbm, src_hbm, dst_hbm, u_hbm, out_hbm,
          acc, srcv, dstv, ubig, *rest):
        bufs = [(rest[2 * i], rest[2 * i + 1]) for i in range(NSETS)]
        zbuf = rest[2 * NSETS]
        semg = rest[2 * NSETS + 1:2 * NSETS + 1 + NSETS]
        sems = rest[2 * NSETS + 1 + NSETS:]

        c = lax.axis_index("c")
        s = lax.axis_index("s")
        w = s * NC + c
        aw = w * 78 + jnp.minimum(w, N_GROUPS - 78 * NW)
        z16 = jnp.zeros((16,), _F32)
        lanes = lax.iota(jnp.int32, 16)
        e0 = jnp.where(lanes == 0, 1.0, 0.0)
        lo_mask = lanes < 8

        def init_body(i, carry):
            for half in range(width // 16):
                zbuf[i, pl.ds(16 * half, 16)] = z16
            if layer1:
                for _, mb in bufs:
                    mb[i, pl.ds(16, 16)] = e0
            return carry

        lax.fori_loop(0, G, init_body, 0)

        base = s * RPS
        for off, cnt in _ZERO_CHUNKS:
            pltpu.sync_copy(zbuf.at[pl.ds(0, cnt)], acc.at[pl.ds(base + off, cnt)])
        plsc.subcore_barrier()

        def make_edge_body(slot, gb, mb):
            def edge_body(j, cc):
                uvec = ubig[pl.ds(slot * G + j * 16, 16)]
                for kk in range(16):
                    i = j * 16 + kk
                    us = uvec[kk]
                    if layer1:
                        g0 = gb[i, pl.ds(0, 16)]
                        gd = gb[i, pl.ds(16, 16)]
                        mb[i, pl.ds(0, 16)] = g0 + us * gd
                    else:
                        row = gb[i, pl.ds(0, 16)]
                        wv = jnp.where(lo_mask, 1.0, us)
                        mb[i, pl.ds(0, 16)] = row * wv
                return cc
            return edge_body

        def iter_body(t, carry):
            g0 = aw + NSETS * t
            pltpu.sync_copy(src_hbm.at[pl.ds(g0, NSETS)], srcv)
            pltpu.sync_copy(dst_hbm.at[pl.ds(g0, NSETS)], dstv)
            pltpu.sync_copy(u_hbm.at[pl.ds(g0 * G, NSETS * G)], ubig)
            dgs = [pltpu.async_copy(tbl_hbm.at[srcv.at[i]], bufs[i][0], semg[i])
                   for i in range(NSETS)]
            dss = []
            for i in range(NSETS):
                gb, mb = bufs[i]
                dgs[i].wait()
                lax.fori_loop(0, G // 16, make_edge_body(i, gb, mb), 0)
                dss.append(pltpu.async_copy(mb, acc.at[dstv.at[i]],
                                            sems[i], add=True))
            for d in dss:
                d.wait()
            return carry

        lax.fori_loop(0, NPAIRS, iter_body, 0)

        def run_group(gidx, slot):
            gb, mb = bufs[slot]
            pltpu.sync_copy(src_hbm.at[gidx], srcv.at[slot])
            dg = pltpu.async_copy(tbl_hbm.at[srcv.at[slot]], gb, semg[slot])
            pltpu.sync_copy(dst_hbm.at[gidx], dstv.at[slot])
            pltpu.sync_copy(u_hbm.at[pl.ds(gidx * G, G)],
                            ubig.at[pl.ds(slot * G, G)])
            dg.wait()
            lax.fori_loop(0, G // 16, make_edge_body(slot, gb, mb), 0)
            pltpu.sync_copy(mb, acc.at[dstv.at[slot]], add=True)

        run_group(aw + NSETS * NPAIRS, 0)
        run_group(aw + NSETS * NPAIRS + 1, 1)

        @pl.when(w < N_GROUPS - 78 * NW)
        def _tail():
            run_group(aw + NSETS * NPAIRS + 2, 2)

        plsc.subcore_barrier()

        for off, cnt in _ZERO_CHUNKS:
            pltpu.sync_copy(acc.at[pl.ds(base + off, cnt)],
                            out_hbm.at[c, pl.ds(base + off, cnt)])

    return k(tbl, src.reshape(N_GROUPS, G), dst.reshape(N_GROUPS, G), u)


def kernel(x, edge_index, edge_attr, W1, root1, b1, W2, root2, b2):
    src = edge_index[0].astype(jnp.int32)
    dst = edge_index[1].astype(jnp.int32)
    u = edge_attr[:, 0].astype(_F32)

    wcat1 = jnp.concatenate([W1[0], W1[1] - W1[0], root1], axis=1)
    g1, r1 = _tc_prep1(x, wcat1)
    acc_p = _sc_edge_pass(g1, src, dst, u, 32, True)

    wg2 = jnp.zeros((16, 16), _F32)
    wg2 = wg2.at[:, 0:7].set(W2[0]).at[:, 8:15].set(W2[1] - W2[0])
    wr2 = jnp.zeros((16, 8), _F32).at[:, 0:7].set(root2)
    b1row = b1.reshape(1, 16).astype(_F32)
    g2, r2, dinv = _tc_mid(acc_p, r1, b1row, wg2, wr2)

    acc2_p = _sc_edge_pass(g2, src, dst, u, 16, False)
    b2row = jnp.zeros((1, 8), _F32).at[0, 0:7].set(b2)
    out8 = _tc_final(acc2_p, r2, dinv, b2row)
    return out8[:, 0:7]


# speedup vs baseline: 17.2835x; 1.1127x over previous
import functools

import jax
import jax.numpy as jnp
from jax import lax
from jax.experimental import pallas as pl
from jax.experimental.pallas import tpu as pltpu
from jax.experimental.pallas import tpu_sc as plsc

N = 10000
E = 320000
G = 128
N_GROUPS = E // G
NC = 2
NS = 16
NW = NC * NS
N_PAD = 10240
RPS = N_PAD // NS

_F32 = jnp.float32


_BLK = 1000


def _prep1_body(x_ref, w_ref, g1_ref, r1_ref):
    y = jnp.dot(x_ref[...], w_ref[...], preferred_element_type=_F32)
    g1_ref[...] = y[:, 0:32]
    r1_ref[...] = y[:, 32:48]


def _tc_prep1(x, wcat1):
    return pl.pallas_call(
        _prep1_body,
        grid=(N // _BLK,),
        in_specs=[
            pl.BlockSpec((_BLK, 128), lambda i: (i, 0)),
            pl.BlockSpec((128, 48), lambda i: (0, 0)),
        ],
        out_specs=[
            pl.BlockSpec((_BLK, 32), lambda i: (i, 0)),
            pl.BlockSpec((_BLK, 16), lambda i: (i, 0)),
        ],
        out_shape=[
            jax.ShapeDtypeStruct((N, 32), _F32),
            jax.ShapeDtypeStruct((N, 16), _F32),
        ],
    )(x, wcat1)


def _mid_body(accp_ref, r1_ref, b1_ref, wg2_ref, wr2_ref, g2_ref, r2_ref, di_ref):
    a = accp_ref[0] + accp_ref[1]
    agg = a[:, 0:16]
    deg = jnp.maximum(a[:, 16:17], 1.0)
    v = agg / deg + r1_ref[...] + b1_ref[...]
    h = jnp.where(v > 0, v, jnp.exp(v) - 1.0)
    g2_ref[...] = jnp.dot(h, wg2_ref[...], preferred_element_type=_F32)
    r2_ref[...] = jnp.dot(h, wr2_ref[...], preferred_element_type=_F32)
    di_ref[...] = jnp.broadcast_to(1.0 / deg, deg.shape[:1] + (8,))


def _tc_mid(acc_p, r1, b1row, wg2, wr2):
    return pl.pallas_call(
        _mid_body,
        grid=(N // _BLK,),
        in_specs=[
            pl.BlockSpec((2, _BLK, 32), lambda i: (0, i, 0)),
            pl.BlockSpec((_BLK, 16), lambda i: (i, 0)),
            pl.BlockSpec((1, 16), lambda i: (0, 0)),
            pl.BlockSpec((16, 16), lambda i: (0, 0)),
            pl.BlockSpec((16, 8), lambda i: (0, 0)),
        ],
        out_specs=[
            pl.BlockSpec((_BLK, 16), lambda i: (i, 0)),
            pl.BlockSpec((_BLK, 8), lambda i: (i, 0)),
            pl.BlockSpec((_BLK, 8), lambda i: (i, 0)),
        ],
        out_shape=[
            jax.ShapeDtypeStruct((N, 16), _F32),
            jax.ShapeDtypeStruct((N, 8), _F32),
            jax.ShapeDtypeStruct((N, 8), _F32),
        ],
    )(acc_p, r1, b1row, wg2, wr2)


def _final_body(acc2_ref, r2_ref, di_ref, b2_ref, o_ref):
    a2 = acc2_ref[0] + acc2_ref[1]
    s = a2[:, 0:8] + a2[:, 8:16]
    s = s * di_ref[...] + r2_ref[...] + b2_ref[...]
    lane = lax.broadcasted_iota(jnp.int32, s.shape, 1)
    mask = lane < 7
    sm = jnp.where(mask, s, -1e30)
    m = jnp.max(sm, axis=1, keepdims=True)
    e = jnp.where(mask, jnp.exp(sm - m), 0.0)
    lse = jnp.log(jnp.sum(e, axis=1, keepdims=True))
    o_ref[...] = s - m - lse


def _tc_final(acc2_p, r2, dinv, b2row):
    return pl.pallas_call(
        _final_body,
        grid=(N // _BLK,),
        in_specs=[
            pl.BlockSpec((2, _BLK, 16), lambda i: (0, i, 0)),
            pl.BlockSpec((_BLK, 8), lambda i: (i, 0)),
            pl.BlockSpec((_BLK, 8), lambda i: (i, 0)),
            pl.BlockSpec((1, 8), lambda i: (0, 0)),
        ],
        out_specs=pl.BlockSpec((_BLK, 8), lambda i: (i, 0)),
        out_shape=jax.ShapeDtypeStruct((N, 8), _F32),
    )(acc2_p, r2, dinv, b2row)


_ZERO_CHUNKS = ((0, 128), (128, 128), (256, 128), (384, 128), (512, 128))

NSETS = 8
NPAIRS = 9


def _sc_edge_pass(tbl, src, dst, u, width, layer1):
    mesh = plsc.VectorSubcoreMesh(core_axis_name="c", subcore_axis_name="s")
    scratch = [
        pltpu.VMEM_SHARED((N_PAD, width), _F32),
        pltpu.VMEM((NSETS, G), jnp.int32),
        pltpu.VMEM((NSETS, G), jnp.int32),
        pltpu.VMEM((NSETS * G,), _F32),
    ]
    for _ in range(NSETS):
        scratch += [
            pltpu.VMEM((G, width), _F32),
            pltpu.VMEM((G, width), _F32),
        ]
    scratch.append(pltpu.VMEM((G, width), _F32))
    scratch += [pltpu.SemaphoreType.DMA] * (2 * NSETS)

    @functools.partial(
        pl.kernel,
        out_type=jax.ShapeDtypeStruct((NC, N_PAD, width), _F32),
        mesh=mesh,
        compiler_params=pltpu.CompilerParams(use_tc_tiling_on_sc=False),
        scratch_types=scratch,
    )
    def k(tbl_hbm, src_hbm, dst_hbm, u_hbm, out_hbm,
          acc, srcv, dstv, ubig, *rest):
        bufs = [(rest[2 * i], rest[2 * i + 1]) for i in range(NSETS)]
        zbuf = rest[2 * NSETS]
        semg = rest[2 * NSETS + 1:2 * NSETS + 1 + NSETS]
        sems = rest[2 * NSETS + 1 + NSETS:]

        c = lax.axis_index("c")
        s = lax.axis_index("s")
        w = s * NC + c
        aw = w * 78 + jnp.minimum(w, N_GROUPS - 78 * NW)
        z16 = jnp.zeros((16,), _F32)
        lanes = lax.iota(jnp.int32, 16)
        e0 = jnp.where(lanes == 0, 1.0, 0.0)
        lo_mask = lanes < 8

        def init_body(i, carry):
            for half in range(width // 16):
                zbuf[i, pl.ds(16 * half, 16)] = z16
            if layer1:
                for _, mb in bufs:
                    mb[i, pl.ds(16, 16)] = e0
            return carry

        lax.fori_loop(0, G, init_body, 0)

        base = s * RPS
        for off, cnt in _ZERO_CHUNKS:
            pltpu.sync_copy(zbuf.at[pl.ds(0, cnt)], acc.at[pl.ds(base + off, cnt)])
        plsc.subcore_barrier()

        def make_edge_body(slot, gb, mb):
            def edge_body(j, cc):
                uvec = ubig[pl.ds(slot * G + j * 16, 16)]
                for kk in range(16):
                    i = j * 16 + kk
                    us = uvec[kk]
                    if layer1:
                        g0 = gb[i, pl.ds(0, 16)]
                        gd = gb[i, pl.ds(16, 16)]
                        mb[i, pl.ds(0, 16)] = g0 + us * gd
                    else:
                        row = gb[i, pl.ds(0, 16)]
                        wv = jnp.where(lo_mask, 1.0, us)
                        mb[i, pl.ds(0, 16)] = row * wv
                return cc
            return edge_body

        def iter_body(t, carry):
            g0 = aw + NSETS * t
            pltpu.sync_copy(src_hbm.at[pl.ds(g0, NSETS)], srcv)
            pltpu.sync_copy(dst_hbm.at[pl.ds(g0, NSETS)], dstv)
            pltpu.sync_copy(u_hbm.at[pl.ds(g0 * G, NSETS * G)], ubig)
            dgs = [pltpu.async_copy(tbl_hbm.at[srcv.at[i]], bufs[i][0], semg[i])
                   for i in range(NSETS)]
            dss = []
            for i in range(NSETS):
                gb, mb = bufs[i]
                dgs[i].wait()
                lax.fori_loop(0, G // 16, make_edge_body(i, gb, mb), 0)
                dss.append(pltpu.async_copy(mb, acc.at[dstv.at[i]],
                                            sems[i], add=True))
            for d in dss:
                d.wait()
            return carry

        lax.fori_loop(0, NPAIRS, iter_body, 0)

        def run_group(gidx, slot):
            gb, mb = bufs[slot]
            pltpu.sync_copy(src_hbm.at[gidx], srcv.at[slot])
            dg = pltpu.async_copy(tbl_hbm.at[srcv.at[slot]], gb, semg[slot])
            pltpu.sync_copy(dst_hbm.at[gidx], dstv.at[slot])
            pltpu.sync_copy(u_hbm.at[pl.ds(gidx * G, G)],
                            ubig.at[pl.ds(slot * G, G)])
            dg.wait()
            lax.fori_loop(0, G // 16, make_edge_body(slot, gb, mb), 0)
            pltpu.sync_copy(mb, acc.at[dstv.at[slot]], add=True)

        for ti in range(78 - NSETS * NPAIRS):
            run_group(aw + NSETS * NPAIRS + ti, ti)

        @pl.when(w < N_GROUPS - 78 * NW)
        def _tail():
            run_group(aw + 78, 78 - NSETS * NPAIRS)

        plsc.subcore_barrier()

        for off, cnt in _ZERO_CHUNKS:
            pltpu.sync_copy(acc.at[pl.ds(base + off, cnt)],
                            out_hbm.at[c, pl.ds(base + off, cnt)])

    return k(tbl, src.reshape(N_GROUPS, G), dst.reshape(N_GROUPS, G), u)


def kernel(x, edge_index, edge_attr, W1, root1, b1, W2, root2, b2):
    src = edge_index[0].astype(jnp.int32)
    dst = edge_index[1].astype(jnp.int32)
    u = edge_attr[:, 0].astype(_F32)

    wcat1 = jnp.concatenate([W1[0], W1[1] - W1[0], root1], axis=1)
    g1, r1 = _tc_prep1(x, wcat1)
    acc_p = _sc_edge_pass(g1, src, dst, u, 32, True)

    wg2 = jnp.zeros((16, 16), _F32)
    wg2 = wg2.at[:, 0:7].set(W2[0]).at[:, 8:15].set(W2[1] - W2[0])
    wr2 = jnp.zeros((16, 8), _F32).at[:, 0:7].set(root2)
    b1row = b1.reshape(1, 16).astype(_F32)
    g2, r2, dinv = _tc_mid(acc_p, r1, b1row, wg2, wr2)

    acc2_p = _sc_edge_pass(g2, src, dst, u, 16, False)
    b2row = jnp.zeros((1, 8), _F32).at[0, 0:7].set(b2)
    out8 = _tc_final(acc2_p, r2, dinv, b2row)
    return out8[:, 0:7]
